# baseline (speedup 1.0000x reference)
import functools
import jax
import jax.numpy as jnp
from jax import lax
from jax.experimental import pallas as pl
from jax.experimental.pallas import tpu as pltpu
from jax.experimental.pallas import tpu_sc as plsc

N_NODES = 10000
BATCH = 2
NT = N_NODES * BATCH
CH = 128
HE = 4
E = 100000
EP = 100352
EPT = EP // 32
EPS = EP // 16
CK = 64
NA = EPT // CK
NB = EPS // CK
NTP = 20480
RPT = NTP // 16
H1 = 32
H2 = 128
NPRED = 9

_mesh = plsc.VectorSubcoreMesh(core_axis_name="c", subcore_axis_name="s")
def _bfly_sum(v, lanes):
    for sft in (8, 4, 2, 1):
        v = v + v.at[lanes ^ sft].get(mode="promise_in_bounds")
    return v


def _bcast_lane(v, h, lanes):
    return v.at[jnp.full((16,), h, jnp.int32)].get(mode="promise_in_bounds")


def _proj_body(x_ref, w_ref, b_ref, o_ref):
    o_ref[0] = (
        jnp.dot(x_ref[...], w_ref[0], preferred_element_type=jnp.float32)
        + b_ref[0, 0][None, :]
    )


def _proj(x, Wall, ball):
    return pl.pallas_call(
        _proj_body,
        grid=(12, 10),
        in_specs=[
            pl.BlockSpec((2000, CH), lambda i, j: (j, 0)),
            pl.BlockSpec((1, CH, 256), lambda i, j: (i, 0, 0)),
            pl.BlockSpec((1, 1, 256), lambda i, j: (i, 0, 0)),
        ],
        out_specs=pl.BlockSpec(
            (1, 2000, 256), lambda i, j: (i // 2, (i % 2) * 10 + j, 0)
        ),
        out_shape=jax.ShapeDtypeStruct((6, 2 * NT, 256), jnp.float32),
    )(x, Wall, ball.reshape(12, 1, 256))


CKA = 32
NAC = EPT // CKA


def _pass_a_body(tbl,
                 src1, dst1, src2, dst2, src3, dst3, attp,
                 ee1, ee2, ee3, den1, den2, den3,
                 att_v, sva, s2va, dva, d2va, svb, s2vb, dvb, d2vb,
                 rdva, rdvb,
                 xl0a, xl1a, xr0a, xr1a, xl0b, xl1b, xr0b, xr1b,
                 eebufa, eebufb, zbuf, den_sp, sema, semb):
    cid = lax.axis_index("c")
    sid = lax.axis_index("s")
    wid = sid * 2 + cid
    pltpu.sync_copy(attp, att_v)

    def zrow(i, _):
        zbuf[i, :] = jnp.zeros((16,), jnp.float32)
        return _

    lax.fori_loop(0, RPT, zrow, None)

    for t, (src, dst, ee, den) in enumerate((
            (src1, dst1, ee1, den1),
            (src2, dst2, ee2, den2),
            (src3, dst3, ee3, den3))):
        lb = (2 * t) * 2 * NT
        rb = (2 * t + 1) * 2 * NT
        pltpu.sync_copy(zbuf, den_sp.at[pl.ds(sid * RPT, RPT)])
        plsc.subcore_barrier()

        attv = [[[att_v[t, c, h, pl.ds(j * 16, 16)] for j in range(4)]
                 for h in range(HE)] for c in range(2)]
        base = wid * EPT

        def issue(k, sv, s2v, dv, d2v, rdv, xl0, xl1, xr0, xr1, sem):
            off = base + k * CKA
            pltpu.sync_copy(src.at[pl.ds(off, CKA)], sv)
            pltpu.sync_copy(dst.at[pl.ds(off, CKA)], rdv)

            def bump(j, _):
                s = sv[pl.ds(j * 16, 16)]
                d = rdv[pl.ds(j * 16, 16)]
                sv[pl.ds(j * 16, 16)] = s + lb
                s2v[pl.ds(j * 16, 16)] = s + (lb + NT)
                dv[pl.ds(j * 16, 16)] = d + rb
                d2v[pl.ds(j * 16, 16)] = d + (rb + NT)
                return _

            lax.fori_loop(0, CKA // 16, bump, None)
            pltpu.async_copy(tbl.at[sv], xl0, sem)
            pltpu.async_copy(tbl.at[s2v], xl1, sem)
            pltpu.async_copy(tbl.at[dv], xr0, sem)
            pltpu.async_copy(tbl.at[d2v], xr1, sem)

        def drain(xl0, xl1, xr0, xr1, sem):
            for buf in (xl0, xl1, xr0, xr1):
                pltpu.make_async_copy(tbl.at[pl.ds(0, CKA)], buf, sem).wait()

        def compute(k, dv, xl0, xl1, xr0, xr1, eebuf):
            off = base + k * CKA
            lanes = lax.iota(jnp.int32, 16)

            def edge(i, _):
                evec = jnp.zeros((16,), jnp.float32)
                for h in range(HE):
                    acc = jnp.zeros((16,), jnp.float32)
                    for c, (xlb, xrb) in enumerate(((xl0, xr0), (xl1, xr1))):
                        for j in range(4):
                            sl = pl.ds(h * 64 + j * 16, 16)
                            z = xlb[i, sl] + xrb[i, sl]
                            zl = jnp.maximum(z, 0.2 * z)
                            acc = acc + zl * attv[c][h][j]
                    red = _bfly_sum(acc, lanes)
                    evec = jnp.where(lanes == h, red, evec)
                eev = jnp.where(lanes < HE, jnp.exp(evec), 0.0)
                vf = jnp.where(off + i < E, 1.0, 0.0)
                eebuf[i, :] = eev * vf
                return _

            lax.fori_loop(0, CKA, edge, None, unroll=4)
            pltpu.sync_copy(eebuf, den_sp.at[dv], add=True)
            pltpu.sync_copy(eebuf, ee.at[pl.ds(off, CKA)])

        issue(0, sva, s2va, dva, d2va, rdva, xl0a, xl1a, xr0a, xr1a, sema)

        def pair(kk, _):
            ka = 2 * kk
            drain(xl0a, xl1a, xr0a, xr1a, sema)
            issue(ka + 1, svb, s2vb, dvb, d2vb, rdvb, xl0b, xl1b, xr0b, xr1b, semb)
            compute(ka, rdva, xl0a, xl1a, xr0a, xr1a, eebufa)
            drain(xl0b, xl1b, xr0b, xr1b, semb)
            issue(ka + 2, sva, s2va, dva, d2va, rdva, xl0a, xl1a, xr0a, xr1a, sema)
            compute(ka + 1, rdvb, xl0b, xl1b, xr0b, xr1b, eebufb)
            return _

        lax.fori_loop(0, NAC // 2, pair, None)
        drain(xl0a, xl1a, xr0a, xr1a, sema)
        plsc.subcore_barrier()
        pltpu.sync_copy(
            den_sp.at[pl.ds(sid * RPT, RPT)],
            den.at[pl.ds(cid * NTP + sid * RPT, RPT)],
        )
        plsc.subcore_barrier()


def _pass_a(tables, edges, attp):
    f32 = jnp.float32
    i32 = jnp.int32
    kfn = pl.kernel(
        _pass_a_body,
        mesh=_mesh,
        compiler_params=pltpu.CompilerParams(use_tc_tiling_on_sc=False),
        out_type=[jax.ShapeDtypeStruct((EP, 16), f32)] * 3
        + [jax.ShapeDtypeStruct((2 * NTP, 16), f32)] * 3,
        scratch_types=[pltpu.VMEM((3, 2, HE, 64), f32)]
        + [pltpu.VMEM((CKA,), i32)] * 10
        + [pltpu.VMEM((CKA, 256), f32)] * 8
        + [pltpu.VMEM((CKA, 16), f32)] * 2
        + [
            pltpu.VMEM((RPT, 16), f32),
            pltpu.VMEM_SHARED((NTP, 16), f32),
            pltpu.SemaphoreType.DMA,
            pltpu.SemaphoreType.DMA,
        ],
    )
    return kfn(tables, *edges, attp)


CKB = 32
NBC = EPS // CKB


def _pass_b_body(tbl,
                 src1, dst1, src2, dst2, src3, dst3,
                 ee1, ee2, ee3, den1, den2, den3, g_out,
                 sva, scva, dva, d2va, svb, scvb, dvb, d2vb,
                 xlba, eeba, d0ba, d1ba, xlbb, eebb, d0bb, d1bb,
                 mbuf, zbuf, acc_sp, sema, semb):
    cid = lax.axis_index("c")
    sid = lax.axis_index("s")

    def zrow(i, _):
        for jj in range(4):
            zbuf[i, pl.ds(jj * 16, 16)] = jnp.zeros((16,), jnp.float32)
        return _

    lax.fori_loop(0, RPT // 5, zrow, None)
    for q in range(5):
        pltpu.sync_copy(zbuf, acc_sp.at[pl.ds(sid * RPT + q * (RPT // 5), RPT // 5)])
    plsc.subcore_barrier()

    for t, (src, dst, ee, den) in enumerate((
            (src1, dst1, ee1, den1),
            (src2, dst2, ee2, den2),
            (src3, dst3, ee3, den3))):
        lb = (2 * t) * 2 * NT
        base = sid * EPS

        def issue(k, sv, scv, dv, d2v, xlb, eeb, d0b, d1b, sem):
            off = base + k * CKB
            pltpu.sync_copy(src.at[pl.ds(off, CKB)], sv)
            pltpu.sync_copy(dst.at[pl.ds(off, CKB)], dv)

            def bump(j, _):
                scv[pl.ds(j * 16, 16)] = sv[pl.ds(j * 16, 16)] + (cid * NT + lb)
                d2v[pl.ds(j * 16, 16)] = dv[pl.ds(j * 16, 16)] + NTP
                return _

            lax.fori_loop(0, CKB // 16, bump, None)
            pltpu.async_copy(tbl.at[scv], xlb, sem)
            pltpu.async_copy(den.at[dv], d0b, sem)
            pltpu.async_copy(den.at[d2v], d1b, sem)
            pltpu.async_copy(ee.at[pl.ds(off, CKB)], eeb, sem)

        def drain(xlb, eeb, d0b, d1b, sem):
            pltpu.make_async_copy(tbl.at[pl.ds(0, CKB)], xlb, sem).wait()
            pltpu.make_async_copy(den.at[pl.ds(0, CKB)], d0b, sem).wait()
            pltpu.make_async_copy(den.at[pl.ds(0, CKB)], d1b, sem).wait()
            pltpu.make_async_copy(ee.at[pl.ds(0, CKB)], eeb, sem).wait()

        def compute(dv, xlb, eeb, d0b, d1b):
            lanes = lax.iota(jnp.int32, 16)

            def edge(i, _):
                dsum = d0b[i, :] + d1b[i, :] + 1e-16
                av = eeb[i, :] * 0.25 / dsum
                a = [_bcast_lane(av, h, lanes) for h in range(HE)]
                for jj in range(4):
                    m = a[0] * xlb[i, pl.ds(jj * 16, 16)]
                    for h in range(1, HE):
                        m = m + a[h] * xlb[i, pl.ds(h * 64 + jj * 16, 16)]
                    mbuf[i, pl.ds(jj * 16, 16)] = m
                return _

            lax.fori_loop(0, CKB, edge, None, unroll=8)
            pltpu.sync_copy(mbuf, acc_sp.at[dv], add=True)

        issue(0, sva, scva, dva, d2va, xlba, eeba, d0ba, d1ba, sema)

        def pair(kk, _):
            ka = 2 * kk
            drain(xlba, eeba, d0ba, d1ba, sema)
            issue(ka + 1, svb, scvb, dvb, d2vb, xlbb, eebb, d0bb, d1bb, semb)
            compute(dva, xlba, eeba, d0ba, d1ba)
            drain(xlbb, eebb, d0bb, d1bb, semb)
            issue(ka + 2, sva, scva, dva, d2va, xlba, eeba, d0ba, d1ba, sema)
            compute(dvb, xlbb, eebb, d0bb, d1bb)
            return _

        lax.fori_loop(0, NBC // 2, pair, None)
        drain(xlba, eeba, d0ba, d1ba, sema)

    plsc.subcore_barrier()
    pltpu.sync_copy(
        acc_sp.at[pl.ds(sid * RPT, RPT)],
        g_out.at[pl.ds(cid * NTP + sid * RPT, RPT)],
    )


def _pass_b(xlts, edges, ees, dens):
    f32 = jnp.float32
    i32 = jnp.int32
    kfn = pl.kernel(
        _pass_b_body,
        mesh=_mesh,
        compiler_params=pltpu.CompilerParams(use_tc_tiling_on_sc=False),
        out_type=jax.ShapeDtypeStruct((2 * NTP, 64), f32),
        scratch_types=[pltpu.VMEM((CKB,), i32)] * 8
        + [
            pltpu.VMEM((CKB, 256), f32),
            pltpu.VMEM((CKB, 16), f32),
            pltpu.VMEM((CKB, 16), f32),
            pltpu.VMEM((CKB, 16), f32),
            pltpu.VMEM((CKB, 256), f32),
            pltpu.VMEM((CKB, 16), f32),
            pltpu.VMEM((CKB, 16), f32),
            pltpu.VMEM((CKB, 16), f32),
            pltpu.VMEM((CKB, 64), f32),
            pltpu.VMEM((RPT // 5, 64), f32),
            pltpu.VMEM_SHARED((NTP, 64), f32),
            pltpu.SemaphoreType.DMA,
            pltpu.SemaphoreType.DMA,
        ],
    )
    return kfn(xlts, *edges, *ees, *dens)


def _x1_body(g0_ref, g1_ref, bs_ref, w_ref, o_ref):
    bsum = bs_ref[0] + bs_ref[1] + bs_ref[2]
    r = jnp.concatenate([g0_ref[...], g1_ref[...]], axis=1) + bsum[None, :]
    r = jnp.maximum(r, 0.0)
    o_ref[0] = lax.dot_general(
        r, w_ref[...], (((0,), (1,)), ((), ())),
        preferred_element_type=jnp.float32,
    )


def _x1(g40, bstack, Wih1):
    return pl.pallas_call(
        _x1_body,
        grid=(BATCH,),
        in_specs=[
            pl.BlockSpec((N_NODES, 64), lambda b: (b, 0)),
            pl.BlockSpec((N_NODES, 64), lambda b: (b, 0)),
            pl.BlockSpec((3, CH), lambda b: (0, 0)),
            pl.BlockSpec((CH, N_NODES), lambda b: (0, 0)),
        ],
        out_specs=pl.BlockSpec((1, CH, CH), lambda b: (b, 0, 0)),
        out_shape=jax.ShapeDtypeStruct((BATCH, CH, CH), jnp.float32),
    )(g40[0:NT], g40[NTP:NTP + NT], bstack, Wih1)


def _sigm(x):
    return 1.0 / (1.0 + jnp.exp(-x))


def _lstm_body(x1_ref, whh1_ref, bih1_ref, bhh1_ref,
               wih2_ref, whh2_ref, bih2_ref, bhh2_ref,
               o_ref, hs1_ref, x2_ref):
    b1 = (bih1_ref[0] + bhh1_ref[0])[None, :]

    def step1(t, hc):
        h, c = hc
        xt = x1_ref[:, pl.ds(t, 1), :].reshape(BATCH, 4 * H1)
        gates = xt + lax.dot_general(
            h, whh1_ref[...], (((1,), (1,)), ((), ())),
            preferred_element_type=jnp.float32) + b1
        i_ = _sigm(gates[:, 0:H1])
        f_ = _sigm(gates[:, H1:2 * H1])
        g_ = jnp.tanh(gates[:, 2 * H1:3 * H1])
        o_ = _sigm(gates[:, 3 * H1:4 * H1])
        c2 = f_ * c + i_ * g_
        h2 = o_ * jnp.tanh(c2)
        hs1_ref[pl.ds(t, 1), :, :] = h2.reshape(1, BATCH, H1)
        return (h2, c2)

    z1 = jnp.zeros((BATCH, H1), jnp.float32)
    lax.fori_loop(0, CH, step1, (z1, z1), unroll=4)

    x2_ref[...] = lax.dot_general(
        hs1_ref[...].reshape(CH * BATCH, H1), wih2_ref[...],
        (((1,), (1,)), ((), ())),
        preferred_element_type=jnp.float32).reshape(CH, BATCH, 4 * H2)

    b2 = (bih2_ref[0] + bhh2_ref[0])[None, :]

    def step2(t, hc):
        h, c = hc
        xt = x2_ref[pl.ds(t, 1), :, :].reshape(BATCH, 4 * H2)
        gates = xt + lax.dot_general(
            h, whh2_ref[...], (((1,), (1,)), ((), ())),
            preferred_element_type=jnp.float32) + b2
        i_ = _sigm(gates[:, 0:H2])
        f_ = _sigm(gates[:, H2:2 * H2])
        g_ = jnp.tanh(gates[:, 2 * H2:3 * H2])
        o_ = _sigm(gates[:, 3 * H2:4 * H2])
        c2 = f_ * c + i_ * g_
        h2 = o_ * jnp.tanh(c2)
        return (h2, c2)

    z2 = jnp.zeros((BATCH, H2), jnp.float32)
    hf, _ = lax.fori_loop(0, CH, step2, (z2, z2), unroll=4)
    o_ref[...] = hf


def _lstm(x1, Whh1, bih1, bhh1, Wih2, Whh2, bih2, bhh2):
    return pl.pallas_call(
        _lstm_body,
        out_shape=jax.ShapeDtypeStruct((BATCH, H2), jnp.float32),
        scratch_shapes=[
            pltpu.VMEM((CH, BATCH, H1), jnp.float32),
            pltpu.VMEM((CH, BATCH, 4 * H2), jnp.float32),
        ],
    )(x1, Whh1, bih1.reshape(1, -1), bhh1.reshape(1, -1),
      Wih2, Whh2, bih2.reshape(1, -1), bhh2.reshape(1, -1))


def _lin_body(h_ref, w_ref, b_ref, o_ref):
    o_ref[0] = lax.dot_general(
        h_ref[...], w_ref[...], (((1,), (1,)), ((), ())),
        preferred_element_type=jnp.float32) + b_ref[0, 0][None, :]


def _linear(hf, Wlin, blin):
    return pl.pallas_call(
        _lin_body,
        grid=(45,),
        in_specs=[
            pl.BlockSpec((BATCH, H2), lambda i: (0, 0)),
            pl.BlockSpec((2000, H2), lambda i: (i, 0)),
            pl.BlockSpec((1, 1, 2000), lambda i: (i, 0, 0)),
        ],
        out_specs=pl.BlockSpec((1, BATCH, 2000), lambda i: (i, 0, 0)),
        out_shape=jax.ShapeDtypeStruct((45, BATCH, 2000), jnp.float32),
    )(hf, Wlin, blin.reshape(45, 1, 2000))


def _chunk_w(W):
    Wp = W.reshape(CH, HE, 2, 64)
    return [Wp[:, :, c, :].reshape(CH, 256) for c in range(2)]


def _chunk_b(b):
    bp = b.reshape(HE, 2, 64)
    return [bp[:, c, :].reshape(256) for c in range(2)]


def _pad_edges(ei):
    z = jnp.zeros((EP + CK - E,), jnp.int32)
    return (jnp.concatenate([ei[0], z]), jnp.concatenate([ei[1], z]))


def kernel(x, edge_index_1, edge_index_2, edge_index_3, Wl1, bl1, Wr1, br1, att1, bias1, Wl2, bl2, Wr2, br2, att2, bias2, Wl3, bl3, Wr3, br3, att3, bias3, Wih1, Whh1, bih1, bhh1, Wih2, Whh2, bih2, bhh2, Wlin, blin):
    Wall, ball = [], []
    for (Wl, bl, Wr, br) in ((Wl1, bl1, Wr1, br1), (Wl2, bl2, Wr2, br2),
                             (Wl3, bl3, Wr3, br3)):
        for (W, b) in ((Wl, bl), (Wr, br)):
            Wall += _chunk_w(W)
            ball += _chunk_b(b)
    Wall = jnp.stack(Wall)
    ball = jnp.stack(ball)
    attp = jnp.stack([a.reshape(HE, 2, 64).transpose(1, 0, 2)
                      for a in (att1, att2, att3)])

    P = _proj(x, Wall, ball)
    s1, d1 = _pad_edges(edge_index_1)
    s2, d2 = _pad_edges(edge_index_2)
    s3, d3 = _pad_edges(edge_index_3)

    Pflat = P.reshape(6 * 2 * NT, 256)
    ee1, ee2, ee3, den1, den2, den3 = _pass_a(
        Pflat, (s1, d1, s2, d2, s3, d3), attp)

    g40 = _pass_b(Pflat, (s1, d1, s2, d2, s3, d3),
                  (ee1, ee2, ee3), (den1, den2, den3))

    bstack = jnp.stack([bias1, bias2, bias3])
    x1 = _x1(g40, bstack, Wih1)
    hf = _lstm(x1, Whh1, bih1, bhh1, Wih2, Whh2, bih2, bhh2)
    out = _linear(hf, Wlin, blin)
    out = out.transpose(1, 0, 2).reshape(BATCH, N_NODES * NPRED)
    return out.reshape(BATCH * N_NODES, NPRED)


# speedup vs baseline: 1.0020x; 1.0020x over previous
import functools
import jax
import jax.numpy as jnp
from jax import lax
from jax.experimental import pallas as pl
from jax.experimental.pallas import tpu as pltpu
from jax.experimental.pallas import tpu_sc as plsc

N_NODES = 10000
BATCH = 2
NT = N_NODES * BATCH
CH = 128
HE = 4
E = 100000
EP = 100352
EPT = EP // 32
EPS = EP // 16
CK = 64
NA = EPT // CK
NB = EPS // CK
NTP = 20480
RPT = NTP // 16
H1 = 32
H2 = 128
NPRED = 9

_mesh = plsc.VectorSubcoreMesh(core_axis_name="c", subcore_axis_name="s")
def _bfly_sum(v, lanes):
    for sft in (8, 4, 2, 1):
        v = v + v.at[lanes ^ sft].get(mode="promise_in_bounds")
    return v


def _bcast_lane(v, h, lanes):
    return v.at[jnp.full((16,), h, jnp.int32)].get(mode="promise_in_bounds")


def _proj_body(x_ref, w_ref, b_ref, o_ref):
    o_ref[0] = (
        jnp.dot(x_ref[...], w_ref[0], preferred_element_type=jnp.float32)
        + b_ref[0, 0][None, :]
    )


def _proj(x, Wall, ball):
    return pl.pallas_call(
        _proj_body,
        grid=(12, 10),
        in_specs=[
            pl.BlockSpec((2000, CH), lambda i, j: (j, 0)),
            pl.BlockSpec((1, CH, 256), lambda i, j: (i, 0, 0)),
            pl.BlockSpec((1, 1, 256), lambda i, j: (i, 0, 0)),
        ],
        out_specs=pl.BlockSpec(
            (1, 2000, 256), lambda i, j: (i // 2, (i % 2) * 10 + j, 0)
        ),
        out_shape=jax.ShapeDtypeStruct((6, 2 * NT, 256), jnp.float32),
    )(x, Wall, ball.reshape(12, 1, 256))


CKA = 32
NAC = EPT // CKA


def _pass_a_body(tbl,
                 src1, dst1, src2, dst2, src3, dst3, attp,
                 ee1, ee2, ee3, den1, den2, den3,
                 att_v, sva, s2va, dva, d2va, svb, s2vb, dvb, d2vb,
                 rdva, rdvb,
                 xl0a, xl1a, xr0a, xr1a, xl0b, xl1b, xr0b, xr1b,
                 eebufa, eebufb, zbuf, den_sp, sema, semb):
    cid = lax.axis_index("c")
    sid = lax.axis_index("s")
    wid = sid * 2 + cid
    pltpu.sync_copy(attp, att_v)

    def zrow(i, _):
        zbuf[i, :] = jnp.zeros((16,), jnp.float32)
        return _

    lax.fori_loop(0, RPT, zrow, None)

    for t, (src, dst, ee, den) in enumerate((
            (src1, dst1, ee1, den1),
            (src2, dst2, ee2, den2),
            (src3, dst3, ee3, den3))):
        lb = (2 * t) * 2 * NT
        rb = (2 * t + 1) * 2 * NT
        pltpu.sync_copy(zbuf, den_sp.at[pl.ds(sid * RPT, RPT)])
        plsc.subcore_barrier()

        attv = [[[att_v[t, c, h, pl.ds(j * 16, 16)] for j in range(4)]
                 for h in range(HE)] for c in range(2)]
        base = wid * EPT

        def issue(k, sv, s2v, dv, d2v, rdv, xl0, xl1, xr0, xr1, sem):
            off = base + k * CKA
            pltpu.sync_copy(src.at[pl.ds(off, CKA)], sv)
            pltpu.sync_copy(dst.at[pl.ds(off, CKA)], rdv)

            def bump(j, _):
                s = sv[pl.ds(j * 16, 16)]
                d = rdv[pl.ds(j * 16, 16)]
                sv[pl.ds(j * 16, 16)] = s + lb
                s2v[pl.ds(j * 16, 16)] = s + (lb + NT)
                dv[pl.ds(j * 16, 16)] = d + rb
                d2v[pl.ds(j * 16, 16)] = d + (rb + NT)
                return _

            lax.fori_loop(0, CKA // 16, bump, None)
            pltpu.async_copy(tbl.at[sv], xl0, sem)
            pltpu.async_copy(tbl.at[s2v], xl1, sem)
            pltpu.async_copy(tbl.at[dv], xr0, sem)
            pltpu.async_copy(tbl.at[d2v], xr1, sem)

        def drain(xl0, xl1, xr0, xr1, sem):
            for buf in (xl0, xl1, xr0, xr1):
                pltpu.make_async_copy(tbl.at[pl.ds(0, CKA)], buf, sem).wait()

        def compute(k, dv, xl0, xl1, xr0, xr1, eebuf):
            off = base + k * CKA
            lanes = lax.iota(jnp.int32, 16)

            def edge(i, _):
                evec = jnp.zeros((16,), jnp.float32)
                for h in range(HE):
                    acc = jnp.zeros((16,), jnp.float32)
                    for c, (xlb, xrb) in enumerate(((xl0, xr0), (xl1, xr1))):
                        for j in range(4):
                            sl = pl.ds(h * 64 + j * 16, 16)
                            z = xlb[i, sl] + xrb[i, sl]
                            zl = jnp.maximum(z, 0.2 * z)
                            acc = acc + zl * attv[c][h][j]
                    red = _bfly_sum(acc, lanes)
                    evec = jnp.where(lanes == h, red, evec)
                eev = jnp.where(lanes < HE, jnp.exp(evec), 0.0)
                vf = jnp.where(off + i < E, 1.0, 0.0)
                eebuf[i, :] = eev * vf
                return _

            lax.fori_loop(0, CKA, edge, None, unroll=2)
            pltpu.sync_copy(eebuf, den_sp.at[dv], add=True)
            pltpu.sync_copy(eebuf, ee.at[pl.ds(off, CKA)])

        issue(0, sva, s2va, dva, d2va, rdva, xl0a, xl1a, xr0a, xr1a, sema)

        def pair(kk, _):
            ka = 2 * kk
            drain(xl0a, xl1a, xr0a, xr1a, sema)
            issue(ka + 1, svb, s2vb, dvb, d2vb, rdvb, xl0b, xl1b, xr0b, xr1b, semb)
            compute(ka, rdva, xl0a, xl1a, xr0a, xr1a, eebufa)
            drain(xl0b, xl1b, xr0b, xr1b, semb)
            issue(ka + 2, sva, s2va, dva, d2va, rdva, xl0a, xl1a, xr0a, xr1a, sema)
            compute(ka + 1, rdvb, xl0b, xl1b, xr0b, xr1b, eebufb)
            return _

        lax.fori_loop(0, NAC // 2, pair, None)
        drain(xl0a, xl1a, xr0a, xr1a, sema)
        plsc.subcore_barrier()
        pltpu.sync_copy(
            den_sp.at[pl.ds(sid * RPT, RPT)],
            den.at[pl.ds(cid * NTP + sid * RPT, RPT)],
        )
        plsc.subcore_barrier()


def _pass_a(tables, edges, attp):
    f32 = jnp.float32
    i32 = jnp.int32
    kfn = pl.kernel(
        _pass_a_body,
        mesh=_mesh,
        compiler_params=pltpu.CompilerParams(use_tc_tiling_on_sc=False),
        out_type=[jax.ShapeDtypeStruct((EP, 16), f32)] * 3
        + [jax.ShapeDtypeStruct((2 * NTP, 16), f32)] * 3,
        scratch_types=[pltpu.VMEM((3, 2, HE, 64), f32)]
        + [pltpu.VMEM((CKA,), i32)] * 10
        + [pltpu.VMEM((CKA, 256), f32)] * 8
        + [pltpu.VMEM((CKA, 16), f32)] * 2
        + [
            pltpu.VMEM((RPT, 16), f32),
            pltpu.VMEM_SHARED((NTP, 16), f32),
            pltpu.SemaphoreType.DMA,
            pltpu.SemaphoreType.DMA,
        ],
    )
    return kfn(tables, *edges, attp)


CKB = 32
NBC = EPS // CKB


def _pass_b_body(tbl,
                 src1, dst1, src2, dst2, src3, dst3,
                 ee1, ee2, ee3, den1, den2, den3, g_out,
                 sva, scva, dva, d2va, svb, scvb, dvb, d2vb,
                 xlba, eeba, d0ba, d1ba, xlbb, eebb, d0bb, d1bb,
                 mbuf, zbuf, acc_sp, sema, semb):
    cid = lax.axis_index("c")
    sid = lax.axis_index("s")

    def zrow(i, _):
        for jj in range(4):
            zbuf[i, pl.ds(jj * 16, 16)] = jnp.zeros((16,), jnp.float32)
        return _

    lax.fori_loop(0, RPT // 5, zrow, None)
    for q in range(5):
        pltpu.sync_copy(zbuf, acc_sp.at[pl.ds(sid * RPT + q * (RPT // 5), RPT // 5)])
    plsc.subcore_barrier()

    for t, (src, dst, ee, den) in enumerate((
            (src1, dst1, ee1, den1),
            (src2, dst2, ee2, den2),
            (src3, dst3, ee3, den3))):
        lb = (2 * t) * 2 * NT
        base = sid * EPS

        def issue(k, sv, scv, dv, d2v, xlb, eeb, d0b, d1b, sem):
            off = base + k * CKB
            pltpu.sync_copy(src.at[pl.ds(off, CKB)], sv)
            pltpu.sync_copy(dst.at[pl.ds(off, CKB)], dv)

            def bump(j, _):
                scv[pl.ds(j * 16, 16)] = sv[pl.ds(j * 16, 16)] + (cid * NT + lb)
                d2v[pl.ds(j * 16, 16)] = dv[pl.ds(j * 16, 16)] + NTP
                return _

            lax.fori_loop(0, CKB // 16, bump, None)
            pltpu.async_copy(tbl.at[scv], xlb, sem)
            pltpu.async_copy(den.at[dv], d0b, sem)
            pltpu.async_copy(den.at[d2v], d1b, sem)
            pltpu.async_copy(ee.at[pl.ds(off, CKB)], eeb, sem)

        def drain(xlb, eeb, d0b, d1b, sem):
            pltpu.make_async_copy(tbl.at[pl.ds(0, CKB)], xlb, sem).wait()
            pltpu.make_async_copy(den.at[pl.ds(0, CKB)], d0b, sem).wait()
            pltpu.make_async_copy(den.at[pl.ds(0, CKB)], d1b, sem).wait()
            pltpu.make_async_copy(ee.at[pl.ds(0, CKB)], eeb, sem).wait()

        def compute(dv, xlb, eeb, d0b, d1b):
            lanes = lax.iota(jnp.int32, 16)

            def edge(i, _):
                dsum = d0b[i, :] + d1b[i, :] + 1e-16
                av = eeb[i, :] * 0.25 / dsum
                a = [_bcast_lane(av, h, lanes) for h in range(HE)]
                for jj in range(4):
                    m = a[0] * xlb[i, pl.ds(jj * 16, 16)]
                    for h in range(1, HE):
                        m = m + a[h] * xlb[i, pl.ds(h * 64 + jj * 16, 16)]
                    mbuf[i, pl.ds(jj * 16, 16)] = m
                return _

            lax.fori_loop(0, CKB, edge, None, unroll=4)
            pltpu.sync_copy(mbuf, acc_sp.at[dv], add=True)

        issue(0, sva, scva, dva, d2va, xlba, eeba, d0ba, d1ba, sema)

        def pair(kk, _):
            ka = 2 * kk
            drain(xlba, eeba, d0ba, d1ba, sema)
            issue(ka + 1, svb, scvb, dvb, d2vb, xlbb, eebb, d0bb, d1bb, semb)
            compute(dva, xlba, eeba, d0ba, d1ba)
            drain(xlbb, eebb, d0bb, d1bb, semb)
            issue(ka + 2, sva, scva, dva, d2va, xlba, eeba, d0ba, d1ba, sema)
            compute(dvb, xlbb, eebb, d0bb, d1bb)
            return _

        lax.fori_loop(0, NBC // 2, pair, None)
        drain(xlba, eeba, d0ba, d1ba, sema)

    plsc.subcore_barrier()
    pltpu.sync_copy(
        acc_sp.at[pl.ds(sid * RPT, RPT)],
        g_out.at[pl.ds(cid * NTP + sid * RPT, RPT)],
    )


def _pass_b(xlts, edges, ees, dens):
    f32 = jnp.float32
    i32 = jnp.int32
    kfn = pl.kernel(
        _pass_b_body,
        mesh=_mesh,
        compiler_params=pltpu.CompilerParams(use_tc_tiling_on_sc=False),
        out_type=jax.ShapeDtypeStruct((2 * NTP, 64), f32),
        scratch_types=[pltpu.VMEM((CKB,), i32)] * 8
        + [
            pltpu.VMEM((CKB, 256), f32),
            pltpu.VMEM((CKB, 16), f32),
            pltpu.VMEM((CKB, 16), f32),
            pltpu.VMEM((CKB, 16), f32),
            pltpu.VMEM((CKB, 256), f32),
            pltpu.VMEM((CKB, 16), f32),
            pltpu.VMEM((CKB, 16), f32),
            pltpu.VMEM((CKB, 16), f32),
            pltpu.VMEM((CKB, 64), f32),
            pltpu.VMEM((RPT // 5, 64), f32),
            pltpu.VMEM_SHARED((NTP, 64), f32),
            pltpu.SemaphoreType.DMA,
            pltpu.SemaphoreType.DMA,
        ],
    )
    return kfn(xlts, *edges, *ees, *dens)


def _x1_body(g0_ref, g1_ref, bs_ref, w_ref, o_ref):
    bsum = bs_ref[0] + bs_ref[1] + bs_ref[2]
    r = jnp.concatenate([g0_ref[...], g1_ref[...]], axis=1) + bsum[None, :]
    r = jnp.maximum(r, 0.0)
    o_ref[0] = lax.dot_general(
        r, w_ref[...], (((0,), (1,)), ((), ())),
        preferred_element_type=jnp.float32,
    )


def _x1(g40, bstack, Wih1):
    return pl.pallas_call(
        _x1_body,
        grid=(BATCH,),
        in_specs=[
            pl.BlockSpec((N_NODES, 64), lambda b: (b, 0)),
            pl.BlockSpec((N_NODES, 64), lambda b: (b, 0)),
            pl.BlockSpec((3, CH), lambda b: (0, 0)),
            pl.BlockSpec((CH, N_NODES), lambda b: (0, 0)),
        ],
        out_specs=pl.BlockSpec((1, CH, CH), lambda b: (b, 0, 0)),
        out_shape=jax.ShapeDtypeStruct((BATCH, CH, CH), jnp.float32),
    )(g40[0:NT], g40[NTP:NTP + NT], bstack, Wih1)


def _sigm(x):
    return 1.0 / (1.0 + jnp.exp(-x))


def _lstm_body(x1_ref, whh1_ref, bih1_ref, bhh1_ref,
               wih2_ref, whh2_ref, bih2_ref, bhh2_ref,
               o_ref, hs1_ref, x2_ref):
    b1 = (bih1_ref[0] + bhh1_ref[0])[None, :]

    def step1(t, hc):
        h, c = hc
        xt = x1_ref[:, pl.ds(t, 1), :].reshape(BATCH, 4 * H1)
        gates = xt + lax.dot_general(
            h, whh1_ref[...], (((1,), (1,)), ((), ())),
            preferred_element_type=jnp.float32) + b1
        i_ = _sigm(gates[:, 0:H1])
        f_ = _sigm(gates[:, H1:2 * H1])
        g_ = jnp.tanh(gates[:, 2 * H1:3 * H1])
        o_ = _sigm(gates[:, 3 * H1:4 * H1])
        c2 = f_ * c + i_ * g_
        h2 = o_ * jnp.tanh(c2)
        hs1_ref[pl.ds(t, 1), :, :] = h2.reshape(1, BATCH, H1)
        return (h2, c2)

    z1 = jnp.zeros((BATCH, H1), jnp.float32)
    lax.fori_loop(0, CH, step1, (z1, z1), unroll=4)

    x2_ref[...] = lax.dot_general(
        hs1_ref[...].reshape(CH * BATCH, H1), wih2_ref[...],
        (((1,), (1,)), ((), ())),
        preferred_element_type=jnp.float32).reshape(CH, BATCH, 4 * H2)

    b2 = (bih2_ref[0] + bhh2_ref[0])[None, :]

    def step2(t, hc):
        h, c = hc
        xt = x2_ref[pl.ds(t, 1), :, :].reshape(BATCH, 4 * H2)
        gates = xt + lax.dot_general(
            h, whh2_ref[...], (((1,), (1,)), ((), ())),
            preferred_element_type=jnp.float32) + b2
        i_ = _sigm(gates[:, 0:H2])
        f_ = _sigm(gates[:, H2:2 * H2])
        g_ = jnp.tanh(gates[:, 2 * H2:3 * H2])
        o_ = _sigm(gates[:, 3 * H2:4 * H2])
        c2 = f_ * c + i_ * g_
        h2 = o_ * jnp.tanh(c2)
        return (h2, c2)

    z2 = jnp.zeros((BATCH, H2), jnp.float32)
    hf, _ = lax.fori_loop(0, CH, step2, (z2, z2), unroll=4)
    o_ref[...] = hf


def _lstm(x1, Whh1, bih1, bhh1, Wih2, Whh2, bih2, bhh2):
    return pl.pallas_call(
        _lstm_body,
        out_shape=jax.ShapeDtypeStruct((BATCH, H2), jnp.float32),
        scratch_shapes=[
            pltpu.VMEM((CH, BATCH, H1), jnp.float32),
            pltpu.VMEM((CH, BATCH, 4 * H2), jnp.float32),
        ],
    )(x1, Whh1, bih1.reshape(1, -1), bhh1.reshape(1, -1),
      Wih2, Whh2, bih2.reshape(1, -1), bhh2.reshape(1, -1))


def _lin_body(h_ref, w_ref, b_ref, o_ref):
    o_ref[0] = lax.dot_general(
        h_ref[...], w_ref[...], (((1,), (1,)), ((), ())),
        preferred_element_type=jnp.float32) + b_ref[0, 0][None, :]


def _linear(hf, Wlin, blin):
    return pl.pallas_call(
        _lin_body,
        grid=(45,),
        in_specs=[
            pl.BlockSpec((BATCH, H2), lambda i: (0, 0)),
            pl.BlockSpec((2000, H2), lambda i: (i, 0)),
            pl.BlockSpec((1, 1, 2000), lambda i: (i, 0, 0)),
        ],
        out_specs=pl.BlockSpec((1, BATCH, 2000), lambda i: (i, 0, 0)),
        out_shape=jax.ShapeDtypeStruct((45, BATCH, 2000), jnp.float32),
    )(hf, Wlin, blin.reshape(45, 1, 2000))


def _chunk_w(W):
    Wp = W.reshape(CH, HE, 2, 64)
    return [Wp[:, :, c, :].reshape(CH, 256) for c in range(2)]


def _chunk_b(b):
    bp = b.reshape(HE, 2, 64)
    return [bp[:, c, :].reshape(256) for c in range(2)]


def _pad_edges(ei):
    z = jnp.zeros((EP + CK - E,), jnp.int32)
    return (jnp.concatenate([ei[0], z]), jnp.concatenate([ei[1], z]))


def kernel(x, edge_index_1, edge_index_2, edge_index_3, Wl1, bl1, Wr1, br1, att1, bias1, Wl2, bl2, Wr2, br2, att2, bias2, Wl3, bl3, Wr3, br3, att3, bias3, Wih1, Whh1, bih1, bhh1, Wih2, Whh2, bih2, bhh2, Wlin, blin):
    Wall, ball = [], []
    for (Wl, bl, Wr, br) in ((Wl1, bl1, Wr1, br1), (Wl2, bl2, Wr2, br2),
                             (Wl3, bl3, Wr3, br3)):
        for (W, b) in ((Wl, bl), (Wr, br)):
            Wall += _chunk_w(W)
            ball += _chunk_b(b)
    Wall = jnp.stack(Wall)
    ball = jnp.stack(ball)
    attp = jnp.stack([a.reshape(HE, 2, 64).transpose(1, 0, 2)
                      for a in (att1, att2, att3)])

    P = _proj(x, Wall, ball)
    s1, d1 = _pad_edges(edge_index_1)
    s2, d2 = _pad_edges(edge_index_2)
    s3, d3 = _pad_edges(edge_index_3)

    Pflat = P.reshape(6 * 2 * NT, 256)
    ee1, ee2, ee3, den1, den2, den3 = _pass_a(
        Pflat, (s1, d1, s2, d2, s3, d3), attp)

    g40 = _pass_b(Pflat, (s1, d1, s2, d2, s3, d3),
                  (ee1, ee2, ee3), (den1, den2, den3))

    bstack = jnp.stack([bias1, bias2, bias3])
    x1 = _x1(g40, bstack, Wih1)
    hf = _lstm(x1, Whh1, bih1, bhh1, Wih2, Whh2, bih2, bhh2)
    out = _linear(hf, Wlin, blin)
    out = out.transpose(1, 0, 2).reshape(BATCH, N_NODES * NPRED)
    return out.reshape(BATCH * N_NODES, NPRED)


# speedup vs baseline: 1.0072x; 1.0051x over previous
import functools
import jax
import jax.numpy as jnp
from jax import lax
from jax.experimental import pallas as pl
from jax.experimental.pallas import tpu as pltpu
from jax.experimental.pallas import tpu_sc as plsc

N_NODES = 10000
BATCH = 2
NT = N_NODES * BATCH
CH = 128
HE = 4
E = 100000
EP = 100352
EPT = EP // 32
EPS = EP // 16
CK = 64
NA = EPT // CK
NB = EPS // CK
NTP = NT
RPT = NTP // 16
H1 = 32
H2 = 128
NPRED = 9

_mesh = plsc.VectorSubcoreMesh(core_axis_name="c", subcore_axis_name="s")
def _bfly_sum(v, lanes):
    for sft in (8, 4, 2, 1):
        v = v + v.at[lanes ^ sft].get(mode="promise_in_bounds")
    return v


def _bcast_lane(v, h, lanes):
    return v.at[jnp.full((16,), h, jnp.int32)].get(mode="promise_in_bounds")


def _proj_body(x_ref, w_ref, b_ref, o_ref):
    o_ref[0] = (
        jnp.dot(x_ref[...], w_ref[0], preferred_element_type=jnp.float32)
        + b_ref[0, 0][None, :]
    )


def _proj(x, Wall, ball):
    return pl.pallas_call(
        _proj_body,
        grid=(12, 10),
        in_specs=[
            pl.BlockSpec((2000, CH), lambda i, j: (j, 0)),
            pl.BlockSpec((1, CH, 256), lambda i, j: (i, 0, 0)),
            pl.BlockSpec((1, 1, 256), lambda i, j: (i, 0, 0)),
        ],
        out_specs=pl.BlockSpec(
            (1, 2000, 256), lambda i, j: (i // 2, (i % 2) * 10 + j, 0)
        ),
        out_shape=jax.ShapeDtypeStruct((6, 2 * NT, 256), jnp.float32),
    )(x, Wall, ball.reshape(12, 1, 256))


CKA = 32
NAC = EPT // CKA


def _pass_a_body(tbl,
                 src1, dst1, src2, dst2, src3, dst3, attp,
                 ee1, ee2, ee3, den1, den2, den3,
                 att_v, sva, s2va, dva, d2va, svb, s2vb, dvb, d2vb,
                 rdva, rdvb,
                 xl0a, xl1a, xr0a, xr1a, xl0b, xl1b, xr0b, xr1b,
                 eebufa, eebufb, zbuf, den_sp, sema, semb):
    cid = lax.axis_index("c")
    sid = lax.axis_index("s")
    wid = sid * 2 + cid
    pltpu.sync_copy(attp, att_v)

    def zrow(i, _):
        zbuf[i, :] = jnp.zeros((16,), jnp.float32)
        return _

    lax.fori_loop(0, RPT, zrow, None)

    for t, (src, dst, ee, den) in enumerate((
            (src1, dst1, ee1, den1),
            (src2, dst2, ee2, den2),
            (src3, dst3, ee3, den3))):
        lb = (2 * t) * 2 * NT
        rb = (2 * t + 1) * 2 * NT
        pltpu.sync_copy(zbuf, den_sp.at[pl.ds(sid * RPT, RPT)])
        plsc.subcore_barrier()

        attv = [[[att_v[t, c, h, pl.ds(j * 16, 16)] for j in range(4)]
                 for h in range(HE)] for c in range(2)]
        base = wid * EPT

        def issue(k, sv, s2v, dv, d2v, rdv, xl0, xl1, xr0, xr1, sem):
            off = base + k * CKA
            pltpu.sync_copy(src.at[pl.ds(off, CKA)], sv)
            pltpu.sync_copy(dst.at[pl.ds(off, CKA)], rdv)

            def bump(j, _):
                s = sv[pl.ds(j * 16, 16)]
                d = rdv[pl.ds(j * 16, 16)]
                sv[pl.ds(j * 16, 16)] = s + lb
                s2v[pl.ds(j * 16, 16)] = s + (lb + NT)
                dv[pl.ds(j * 16, 16)] = d + rb
                d2v[pl.ds(j * 16, 16)] = d + (rb + NT)
                return _

            lax.fori_loop(0, CKA // 16, bump, None)
            pltpu.async_copy(tbl.at[sv], xl0, sem)
            pltpu.async_copy(tbl.at[s2v], xl1, sem)
            pltpu.async_copy(tbl.at[dv], xr0, sem)
            pltpu.async_copy(tbl.at[d2v], xr1, sem)

        def drain(xl0, xl1, xr0, xr1, sem):
            for buf in (xl0, xl1, xr0, xr1):
                pltpu.make_async_copy(tbl.at[pl.ds(0, CKA)], buf, sem).wait()

        def compute(k, dv, xl0, xl1, xr0, xr1, eebuf):
            off = base + k * CKA
            lanes = lax.iota(jnp.int32, 16)

            def edge(i, _):
                evec = jnp.zeros((16,), jnp.float32)
                for h in range(HE):
                    acc = jnp.zeros((16,), jnp.float32)
                    for c, (xlb, xrb) in enumerate(((xl0, xr0), (xl1, xr1))):
                        for j in range(4):
                            sl = pl.ds(h * 64 + j * 16, 16)
                            z = xlb[i, sl] + xrb[i, sl]
                            zl = jnp.maximum(z, 0.2 * z)
                            acc = acc + zl * attv[c][h][j]
                    red = _bfly_sum(acc, lanes)
                    evec = jnp.where(lanes == h, red, evec)
                eev = jnp.where(lanes < HE, jnp.exp(evec), 0.0)
                vf = jnp.where(off + i < E, 1.0, 0.0)
                eebuf[i, :] = eev * vf
                return _

            lax.fori_loop(0, CKA, edge, None, unroll=2)
            pltpu.sync_copy(eebuf, den_sp.at[dv], add=True)
            pltpu.sync_copy(eebuf, ee.at[pl.ds(off, CKA)])

        issue(0, sva, s2va, dva, d2va, rdva, xl0a, xl1a, xr0a, xr1a, sema)

        def pair(kk, _):
            ka = 2 * kk
            drain(xl0a, xl1a, xr0a, xr1a, sema)
            issue(ka + 1, svb, s2vb, dvb, d2vb, rdvb, xl0b, xl1b, xr0b, xr1b, semb)
            compute(ka, rdva, xl0a, xl1a, xr0a, xr1a, eebufa)
            drain(xl0b, xl1b, xr0b, xr1b, semb)
            issue(ka + 2, sva, s2va, dva, d2va, rdva, xl0a, xl1a, xr0a, xr1a, sema)
            compute(ka + 1, rdvb, xl0b, xl1b, xr0b, xr1b, eebufb)
            return _

        lax.fori_loop(0, NAC // 2, pair, None)
        drain(xl0a, xl1a, xr0a, xr1a, sema)
        plsc.subcore_barrier()
        pltpu.sync_copy(
            den_sp.at[pl.ds(sid * RPT, RPT)],
            den.at[pl.ds(cid * NTP + sid * RPT, RPT)],
        )
        plsc.subcore_barrier()


def _pass_a(tables, edges, attp):
    f32 = jnp.float32
    i32 = jnp.int32
    kfn = pl.kernel(
        _pass_a_body,
        mesh=_mesh,
        compiler_params=pltpu.CompilerParams(use_tc_tiling_on_sc=False),
        out_type=[jax.ShapeDtypeStruct((EP, 16), f32)] * 3
        + [jax.ShapeDtypeStruct((2 * NTP, 16), f32)] * 3,
        scratch_types=[pltpu.VMEM((3, 2, HE, 64), f32)]
        + [pltpu.VMEM((CKA,), i32)] * 10
        + [pltpu.VMEM((CKA, 256), f32)] * 8
        + [pltpu.VMEM((CKA, 16), f32)] * 2
        + [
            pltpu.VMEM((RPT, 16), f32),
            pltpu.VMEM_SHARED((NTP, 16), f32),
            pltpu.SemaphoreType.DMA,
            pltpu.SemaphoreType.DMA,
        ],
    )
    return kfn(tables, *edges, attp)


CKB = 32
NBC = EPS // CKB


def _pass_b_body(tbl,
                 src1, dst1, src2, dst2, src3, dst3,
                 ee1, ee2, ee3, den1, den2, den3, g_out,
                 sva, scva, dva, d2va, svb, scvb, dvb, d2vb,
                 xlba, eeba, d0ba, d1ba, xlbb, eebb, d0bb, d1bb,
                 mbuf, zbuf, acc_sp, sema, semb):
    cid = lax.axis_index("c")
    sid = lax.axis_index("s")

    def zrow(i, _):
        for jj in range(4):
            zbuf[i, pl.ds(jj * 16, 16)] = jnp.zeros((16,), jnp.float32)
        return _

    lax.fori_loop(0, RPT // 5, zrow, None)
    for q in range(5):
        pltpu.sync_copy(zbuf, acc_sp.at[pl.ds(sid * RPT + q * (RPT // 5), RPT // 5)])
    plsc.subcore_barrier()

    for t, (src, dst, ee, den) in enumerate((
            (src1, dst1, ee1, den1),
            (src2, dst2, ee2, den2),
            (src3, dst3, ee3, den3))):
        lb = (2 * t) * 2 * NT
        base = sid * EPS

        def issue(k, sv, scv, dv, d2v, xlb, eeb, d0b, d1b, sem):
            off = base + k * CKB
            pltpu.sync_copy(src.at[pl.ds(off, CKB)], sv)
            pltpu.sync_copy(dst.at[pl.ds(off, CKB)], dv)

            def bump(j, _):
                scv[pl.ds(j * 16, 16)] = sv[pl.ds(j * 16, 16)] + (cid * NT + lb)
                d2v[pl.ds(j * 16, 16)] = dv[pl.ds(j * 16, 16)] + NTP
                return _

            lax.fori_loop(0, CKB // 16, bump, None)
            pltpu.async_copy(tbl.at[scv], xlb, sem)
            pltpu.async_copy(den.at[dv], d0b, sem)
            pltpu.async_copy(den.at[d2v], d1b, sem)
            pltpu.async_copy(ee.at[pl.ds(off, CKB)], eeb, sem)

        def drain(xlb, eeb, d0b, d1b, sem):
            pltpu.make_async_copy(tbl.at[pl.ds(0, CKB)], xlb, sem).wait()
            pltpu.make_async_copy(den.at[pl.ds(0, CKB)], d0b, sem).wait()
            pltpu.make_async_copy(den.at[pl.ds(0, CKB)], d1b, sem).wait()
            pltpu.make_async_copy(ee.at[pl.ds(0, CKB)], eeb, sem).wait()

        def compute(dv, xlb, eeb, d0b, d1b):
            lanes = lax.iota(jnp.int32, 16)

            def edge(i, _):
                dsum = d0b[i, :] + d1b[i, :] + 1e-16
                av = eeb[i, :] * 0.25 / dsum
                a = [_bcast_lane(av, h, lanes) for h in range(HE)]
                for jj in range(4):
                    m = a[0] * xlb[i, pl.ds(jj * 16, 16)]
                    for h in range(1, HE):
                        m = m + a[h] * xlb[i, pl.ds(h * 64 + jj * 16, 16)]
                    mbuf[i, pl.ds(jj * 16, 16)] = m
                return _

            lax.fori_loop(0, CKB, edge, None, unroll=4)
            pltpu.sync_copy(mbuf, acc_sp.at[dv], add=True)

        issue(0, sva, scva, dva, d2va, xlba, eeba, d0ba, d1ba, sema)

        def pair(kk, _):
            ka = 2 * kk
            drain(xlba, eeba, d0ba, d1ba, sema)
            issue(ka + 1, svb, scvb, dvb, d2vb, xlbb, eebb, d0bb, d1bb, semb)
            compute(dva, xlba, eeba, d0ba, d1ba)
            drain(xlbb, eebb, d0bb, d1bb, semb)
            issue(ka + 2, sva, scva, dva, d2va, xlba, eeba, d0ba, d1ba, sema)
            compute(dvb, xlbb, eebb, d0bb, d1bb)
            return _

        lax.fori_loop(0, NBC // 2, pair, None)
        drain(xlba, eeba, d0ba, d1ba, sema)

    plsc.subcore_barrier()
    pltpu.sync_copy(
        acc_sp.at[pl.ds(sid * RPT, RPT)],
        g_out.at[pl.ds(cid * NTP + sid * RPT, RPT)],
    )


def _pass_b(xlts, edges, ees, dens):
    f32 = jnp.float32
    i32 = jnp.int32
    kfn = pl.kernel(
        _pass_b_body,
        mesh=_mesh,
        compiler_params=pltpu.CompilerParams(use_tc_tiling_on_sc=False),
        out_type=jax.ShapeDtypeStruct((2 * NTP, 64), f32),
        scratch_types=[pltpu.VMEM((CKB,), i32)] * 8
        + [
            pltpu.VMEM((CKB, 256), f32),
            pltpu.VMEM((CKB, 16), f32),
            pltpu.VMEM((CKB, 16), f32),
            pltpu.VMEM((CKB, 16), f32),
            pltpu.VMEM((CKB, 256), f32),
            pltpu.VMEM((CKB, 16), f32),
            pltpu.VMEM((CKB, 16), f32),
            pltpu.VMEM((CKB, 16), f32),
            pltpu.VMEM((CKB, 64), f32),
            pltpu.VMEM((RPT // 5, 64), f32),
            pltpu.VMEM_SHARED((NTP, 64), f32),
            pltpu.SemaphoreType.DMA,
            pltpu.SemaphoreType.DMA,
        ],
    )
    return kfn(xlts, *edges, *ees, *dens)


def _x1_body(g0_ref, g1_ref, bs_ref, w_ref, o_ref):
    bsum = bs_ref[0] + bs_ref[1] + bs_ref[2]
    r = jnp.concatenate([g0_ref[...], g1_ref[...]], axis=1) + bsum[None, :]
    r = jnp.maximum(r, 0.0)
    o_ref[0] = lax.dot_general(
        r, w_ref[...], (((0,), (1,)), ((), ())),
        preferred_element_type=jnp.float32,
    )


def _x1(g40, bstack, Wih1):
    return pl.pallas_call(
        _x1_body,
        grid=(BATCH,),
        in_specs=[
            pl.BlockSpec((N_NODES, 64), lambda b: (b, 0)),
            pl.BlockSpec((N_NODES, 64), lambda b: (2 + b, 0)),
            pl.BlockSpec((3, CH), lambda b: (0, 0)),
            pl.BlockSpec((CH, N_NODES), lambda b: (0, 0)),
        ],
        out_specs=pl.BlockSpec((1, CH, CH), lambda b: (b, 0, 0)),
        out_shape=jax.ShapeDtypeStruct((BATCH, CH, CH), jnp.float32),
    )(g40, g40, bstack, Wih1)


def _sigm(x):
    return 1.0 / (1.0 + jnp.exp(-x))


def _lstm_body(x1_ref, whh1_ref, bih1_ref, bhh1_ref,
               wih2_ref, whh2_ref, bih2_ref, bhh2_ref,
               o_ref, hs1_ref, x2_ref):
    b1 = (bih1_ref[0] + bhh1_ref[0])[None, :]

    def step1(t, hc):
        h, c = hc
        xt = x1_ref[:, pl.ds(t, 1), :].reshape(BATCH, 4 * H1)
        gates = xt + lax.dot_general(
            h, whh1_ref[...], (((1,), (1,)), ((), ())),
            preferred_element_type=jnp.float32) + b1
        i_ = _sigm(gates[:, 0:H1])
        f_ = _sigm(gates[:, H1:2 * H1])
        g_ = jnp.tanh(gates[:, 2 * H1:3 * H1])
        o_ = _sigm(gates[:, 3 * H1:4 * H1])
        c2 = f_ * c + i_ * g_
        h2 = o_ * jnp.tanh(c2)
        hs1_ref[pl.ds(t, 1), :, :] = h2.reshape(1, BATCH, H1)
        return (h2, c2)

    z1 = jnp.zeros((BATCH, H1), jnp.float32)
    lax.fori_loop(0, CH, step1, (z1, z1), unroll=4)

    x2_ref[...] = lax.dot_general(
        hs1_ref[...].reshape(CH * BATCH, H1), wih2_ref[...],
        (((1,), (1,)), ((), ())),
        preferred_element_type=jnp.float32).reshape(CH, BATCH, 4 * H2)

    b2 = (bih2_ref[0] + bhh2_ref[0])[None, :]

    def step2(t, hc):
        h, c = hc
        xt = x2_ref[pl.ds(t, 1), :, :].reshape(BATCH, 4 * H2)
        gates = xt + lax.dot_general(
            h, whh2_ref[...], (((1,), (1,)), ((), ())),
            preferred_element_type=jnp.float32) + b2
        i_ = _sigm(gates[:, 0:H2])
        f_ = _sigm(gates[:, H2:2 * H2])
        g_ = jnp.tanh(gates[:, 2 * H2:3 * H2])
        o_ = _sigm(gates[:, 3 * H2:4 * H2])
        c2 = f_ * c + i_ * g_
        h2 = o_ * jnp.tanh(c2)
        return (h2, c2)

    z2 = jnp.zeros((BATCH, H2), jnp.float32)
    hf, _ = lax.fori_loop(0, CH, step2, (z2, z2), unroll=4)
    o_ref[...] = hf


def _lstm(x1, Whh1, bih1, bhh1, Wih2, Whh2, bih2, bhh2):
    return pl.pallas_call(
        _lstm_body,
        out_shape=jax.ShapeDtypeStruct((BATCH, H2), jnp.float32),
        scratch_shapes=[
            pltpu.VMEM((CH, BATCH, H1), jnp.float32),
            pltpu.VMEM((CH, BATCH, 4 * H2), jnp.float32),
        ],
    )(x1, Whh1, bih1.reshape(1, -1), bhh1.reshape(1, -1),
      Wih2, Whh2, bih2.reshape(1, -1), bhh2.reshape(1, -1))


def _lin_body(h_ref, w_ref, b_ref, o_ref):
    o_ref[0] = lax.dot_general(
        h_ref[...], w_ref[...], (((1,), (1,)), ((), ())),
        preferred_element_type=jnp.float32) + b_ref[0, 0][None, :]


def _linear(hf, Wlin, blin):
    return pl.pallas_call(
        _lin_body,
        grid=(45,),
        in_specs=[
            pl.BlockSpec((BATCH, H2), lambda i: (0, 0)),
            pl.BlockSpec((2000, H2), lambda i: (i, 0)),
            pl.BlockSpec((1, 1, 2000), lambda i: (i, 0, 0)),
        ],
        out_specs=pl.BlockSpec((1, BATCH, 2000), lambda i: (i, 0, 0)),
        out_shape=jax.ShapeDtypeStruct((45, BATCH, 2000), jnp.float32),
    )(hf, Wlin, blin.reshape(45, 1, 2000))


def _chunk_w(W):
    Wp = W.reshape(CH, HE, 2, 64)
    return [Wp[:, :, c, :].reshape(CH, 256) for c in range(2)]


def _chunk_b(b):
    bp = b.reshape(HE, 2, 64)
    return [bp[:, c, :].reshape(256) for c in range(2)]


def _pad_edges(ei):
    z = jnp.zeros((EP + CK - E,), jnp.int32)
    return (jnp.concatenate([ei[0], z]), jnp.concatenate([ei[1], z]))


def kernel(x, edge_index_1, edge_index_2, edge_index_3, Wl1, bl1, Wr1, br1, att1, bias1, Wl2, bl2, Wr2, br2, att2, bias2, Wl3, bl3, Wr3, br3, att3, bias3, Wih1, Whh1, bih1, bhh1, Wih2, Whh2, bih2, bhh2, Wlin, blin):
    Wall, ball = [], []
    for (Wl, bl, Wr, br) in ((Wl1, bl1, Wr1, br1), (Wl2, bl2, Wr2, br2),
                             (Wl3, bl3, Wr3, br3)):
        for (W, b) in ((Wl, bl), (Wr, br)):
            Wall += _chunk_w(W)
            ball += _chunk_b(b)
    Wall = jnp.stack(Wall)
    ball = jnp.stack(ball)
    attp = jnp.stack([a.reshape(HE, 2, 64).transpose(1, 0, 2)
                      for a in (att1, att2, att3)])

    P = _proj(x, Wall, ball)
    s1, d1 = _pad_edges(edge_index_1)
    s2, d2 = _pad_edges(edge_index_2)
    s3, d3 = _pad_edges(edge_index_3)

    Pflat = P.reshape(6 * 2 * NT, 256)
    ee1, ee2, ee3, den1, den2, den3 = _pass_a(
        Pflat, (s1, d1, s2, d2, s3, d3), attp)

    g40 = _pass_b(Pflat, (s1, d1, s2, d2, s3, d3),
                  (ee1, ee2, ee3), (den1, den2, den3))

    bstack = jnp.stack([bias1, bias2, bias3])
    x1 = _x1(g40, bstack, Wih1)
    hf = _lstm(x1, Whh1, bih1, bhh1, Wih2, Whh2, bih2, bhh2)
    out = _linear(hf, Wlin, blin)
    out = out.transpose(1, 0, 2).reshape(BATCH, N_NODES * NPRED)
    return out.reshape(BATCH * N_NODES, NPRED)
